# baseline (speedup 1.0000x reference)
import functools

import jax
import jax.numpy as jnp
from jax import lax
from jax.experimental import pallas as pl
from jax.experimental.pallas import tpu as pltpu
from jax.experimental.pallas import tpu_sc as plsc

TOKENS = 16384
EMBED = 2048
NUM_EXPERTS = 64
ACTIVE_EXPERTS = 8

BLOCK_T = 2048

_NC, _NS, _L = 2, 16, 16
_NW = _NC * _NS
_C = TOKENS // _NW


def _scores_kernel(x_ref, w_ref, b_ref, s_ref):
    s = jax.lax.dot_general(
        x_ref[...], w_ref[...], (((1,), (1,)), ((), ())),
        preferred_element_type=jnp.float32,
    )
    s_ref[...] = s + b_ref[...]


def _merge16(ka, ia, kb, ib):
    kbr = lax.rev(kb, (0,))
    ibr = lax.rev(ib, (0,))
    sel = (ka > kbr) | ((ka == kbr) & (ia < ibr))
    mk = jnp.where(sel, ka, kbr)
    mi = jnp.where(sel, ia, ibr)
    return plsc.sort_key_val(mk, mi, descending=True)


def _sc_router_impl(s_hbm, out_hbm, idx_hbm, sin, sout, sidx):
    wid = lax.axis_index("s") * _NC + lax.axis_index("c")
    base = wid * _C
    pltpu.sync_copy(s_hbm.at[pl.ds(base, _C)], sin)

    iot = lax.iota(jnp.int32, 16)
    lane8 = iot < 8

    def one_token(t):
        k0, i0 = plsc.sort_key_val(sin[t, 0], iot, descending=True)
        k1, i1 = plsc.sort_key_val(sin[t, 1], iot + 16, descending=True)
        k2, i2 = plsc.sort_key_val(sin[t, 2], iot + 32, descending=True)
        k3, i3 = plsc.sort_key_val(sin[t, 3], iot + 48, descending=True)
        ka, ia = _merge16(k0, i0, k1, i1)
        kb, ib = _merge16(k2, i2, k3, i3)
        kt, it = _merge16(ka, ia, kb, ib)

        ev = jnp.where(lane8, jnp.exp(kt), 0.0)
        denom = jnp.sum(ev) + jnp.float32(NUM_EXPERTS - ACTIVE_EXPERTS)
        bb = jnp.full((16,), 1.0, jnp.float32) / denom
        sout[t, 0] = bb
        sout[t, 1] = bb
        sout[t, 2] = bb
        sout[t, 3] = bb
        tvec = jnp.full((16,), 0, jnp.int32) + t
        plsc.store_scatter(
            sout,
            [tvec, lax.shift_right_logical(it, 4), it & 15],
            ev / denom,
            mask=lane8,
        )
        sidx[t] = it

    UNROLL = 8

    def body(g, carry):
        for dt in range(UNROLL):
            one_token(g * UNROLL + dt)
        return carry

    lax.fori_loop(0, _C // UNROLL, body, 0)

    pltpu.sync_copy(sout, out_hbm.at[pl.ds(base, _C)])
    pltpu.sync_copy(sidx, idx_hbm.at[pl.ds(base, _C)])


@functools.lru_cache(maxsize=1)
def _get_sc_router():
    return pl.kernel(
        _sc_router_impl,
        mesh=plsc.VectorSubcoreMesh(core_axis_name="c", subcore_axis_name="s"),
        out_type=[
            jax.ShapeDtypeStruct((TOKENS, 4, 16), jnp.float32),
            jax.ShapeDtypeStruct((TOKENS, 16), jnp.int32),
        ],
        scratch_types=[
            pltpu.VMEM((_C, 4, 16), jnp.float32),
            pltpu.VMEM((_C, 4, 16), jnp.float32),
            pltpu.VMEM((_C, 16), jnp.int32),
        ],
        compiler_params=pltpu.CompilerParams(needs_layout_passes=False, use_tc_tiling_on_sc=False),
    )


@jax.jit
def kernel(inputs, W, b):
    b2 = b.reshape(1, NUM_EXPERTS)
    grid = (TOKENS // BLOCK_T,)
    scores = pl.pallas_call(
        _scores_kernel,
        grid=grid,
        in_specs=[
            pl.BlockSpec((BLOCK_T, EMBED), lambda i: (i, 0)),
            pl.BlockSpec((NUM_EXPERTS, EMBED), lambda i: (0, 0)),
            pl.BlockSpec((1, NUM_EXPERTS), lambda i: (0, 0)),
        ],
        out_specs=pl.BlockSpec((BLOCK_T, NUM_EXPERTS), lambda i: (i, 0)),
        out_shape=jax.ShapeDtypeStruct((TOKENS, NUM_EXPERTS), jnp.float32),
    )(inputs, W, b2)
    out3, idx16 = _get_sc_router()(scores.reshape(TOKENS, 4, 16))
    return (out3.reshape(TOKENS, NUM_EXPERTS), idx16[:, :ACTIVE_EXPERTS])


# speedup vs baseline: 2.5328x; 2.5328x over previous
import jax
import jax.numpy as jnp
from jax.experimental import pallas as pl
from jax.experimental.pallas import tpu as pltpu

TOKENS = 16384
EMBED = 2048
NUM_EXPERTS = 64
ACTIVE_EXPERTS = 8

BLOCK_T = 2048
BLOCK_E = 512
NJ = EMBED // BLOCK_E

_NEG = -1e30


def _router_kernel(x_ref, w_ref, b_ref, out_ref, idx_ref, acc_ref):
    j = pl.program_id(1)
    part = jax.lax.dot_general(
        w_ref[...], x_ref[...], (((1,), (1,)), ((), ())),
        preferred_element_type=jnp.float32,
    )

    @pl.when(j == 0)
    def _():
        acc_ref[...] = part + b_ref[...]

    @pl.when(j > 0)
    def _():
        acc_ref[...] += part

    @pl.when(j == NJ - 1)
    def _():
        st = acc_ref[...]
        iota = jax.lax.broadcasted_iota(jnp.int32, st.shape, 0)
        work = st
        idx_rows = []
        for _ in range(ACTIVE_EXPERTS):
            m = jnp.max(work, axis=0, keepdims=True)
            idx = jnp.min(
                jnp.where(work == m, iota, NUM_EXPERTS), axis=0, keepdims=True
            )
            work = jnp.where(iota == idx, _NEG, work)
            idx_rows.append(idx)

        mask = jnp.where(work == _NEG, st, 0.0)
        mx = jnp.max(mask, axis=0, keepdims=True)
        e = jnp.exp(mask - mx)
        sm = e / jnp.sum(e, axis=0, keepdims=True)
        out_ref[...] = sm.T
        idx_ref[...] = jnp.concatenate(idx_rows, axis=0).T


@jax.jit
def kernel(inputs, W, b):
    b2 = b.reshape(NUM_EXPERTS, 1)
    grid = (TOKENS // BLOCK_T, NJ)
    out, idx = pl.pallas_call(
        _router_kernel,
        grid=grid,
        in_specs=[
            pl.BlockSpec((BLOCK_T, BLOCK_E), lambda i, j: (i, j)),
            pl.BlockSpec((NUM_EXPERTS, BLOCK_E), lambda i, j: (0, j)),
            pl.BlockSpec((NUM_EXPERTS, 1), lambda i, j: (0, 0)),
        ],
        out_specs=[
            pl.BlockSpec((BLOCK_T, NUM_EXPERTS), lambda i, j: (i, 0)),
            pl.BlockSpec((BLOCK_T, ACTIVE_EXPERTS), lambda i, j: (i, 0)),
        ],
        out_shape=[
            jax.ShapeDtypeStruct((TOKENS, NUM_EXPERTS), jnp.float32),
            jax.ShapeDtypeStruct((TOKENS, ACTIVE_EXPERTS), jnp.int32),
        ],
        scratch_shapes=[pltpu.VMEM((NUM_EXPERTS, BLOCK_T), jnp.float32)],
    )(inputs, W, b2)
    return (out, idx)


# speedup vs baseline: 3.3657x; 1.3289x over previous
import jax
import jax.numpy as jnp
from jax.experimental import pallas as pl

TOKENS = 16384
EMBED = 2048
NUM_EXPERTS = 64
ACTIVE_EXPERTS = 8

BLOCK_T = 2048

_NEG = -1e30


def _router_kernel(x_ref, w_ref, b_ref, out_ref, idx_ref):
    x = x_ref[...]
    w = w_ref[...]
    st = jax.lax.dot_general(
        w, x, (((1,), (1,)), ((), ())), preferred_element_type=jnp.float32
    )
    st = st + b_ref[...]

    iota = jax.lax.broadcasted_iota(jnp.int32, st.shape, 0)
    work = st
    chosen = jnp.zeros(st.shape, dtype=jnp.bool_)
    idx_rows = []
    for _ in range(ACTIVE_EXPERTS):
        m = jnp.max(work, axis=0, keepdims=True)
        idx = jnp.min(
            jnp.where(work == m, iota, NUM_EXPERTS), axis=0, keepdims=True
        )
        hit = iota == idx
        work = jnp.where(hit, _NEG, work)
        chosen = jnp.logical_or(chosen, hit)
        idx_rows.append(idx)

    mask = jnp.where(chosen, st, 0.0)
    mx = jnp.max(mask, axis=0, keepdims=True)
    e = jnp.exp(mask - mx)
    sm = e / jnp.sum(e, axis=0, keepdims=True)
    out_ref[...] = sm.T
    idx_ref[...] = jnp.concatenate(idx_rows, axis=0).T


@jax.jit
def kernel(inputs, W, b):
    b2 = b.reshape(NUM_EXPERTS, 1)
    grid = (TOKENS // BLOCK_T,)
    out, idx = pl.pallas_call(
        _router_kernel,
        grid=grid,
        in_specs=[
            pl.BlockSpec((BLOCK_T, EMBED), lambda i: (i, 0)),
            pl.BlockSpec((NUM_EXPERTS, EMBED), lambda i: (0, 0)),
            pl.BlockSpec((NUM_EXPERTS, 1), lambda i: (0, 0)),
        ],
        out_specs=[
            pl.BlockSpec((BLOCK_T, NUM_EXPERTS), lambda i: (i, 0)),
            pl.BlockSpec((BLOCK_T, ACTIVE_EXPERTS), lambda i: (i, 0)),
        ],
        out_shape=[
            jax.ShapeDtypeStruct((TOKENS, NUM_EXPERTS), jnp.float32),
            jax.ShapeDtypeStruct((TOKENS, ACTIVE_EXPERTS), jnp.int32),
        ],
    )(inputs, W, b2)
    return (out, idx)


# speedup vs baseline: 3.3954x; 1.0088x over previous
import jax
import jax.numpy as jnp
from jax.experimental import pallas as pl

TOKENS = 16384
EMBED = 2048
NUM_EXPERTS = 64
ACTIVE_EXPERTS = 8

BLOCK_T = 2048

_NEG = -1e30


def _router_kernel(x_ref, w_ref, b_ref, out_ref, idx_ref):
    x = x_ref[...]
    w = w_ref[...]
    st = jax.lax.dot_general(
        w, x, (((1,), (1,)), ((), ())), preferred_element_type=jnp.float32
    )
    st = st + b_ref[...]

    iota = jax.lax.broadcasted_iota(jnp.int32, st.shape, 0)
    work = st
    idx_rows = []
    for _ in range(ACTIVE_EXPERTS):
        m = jnp.max(work, axis=0, keepdims=True)
        idx = jnp.min(
            jnp.where(work == m, iota, NUM_EXPERTS), axis=0, keepdims=True
        )
        work = jnp.where(iota == idx, _NEG, work)
        idx_rows.append(idx)

    mask = jnp.where(work == _NEG, st, 0.0)
    mx = jnp.max(mask, axis=0, keepdims=True)
    e = jnp.exp(mask - mx)
    sm = e / jnp.sum(e, axis=0, keepdims=True)
    out_ref[...] = sm.T
    idx_ref[...] = jnp.concatenate(idx_rows, axis=0).T


@jax.jit
def kernel(inputs, W, b):
    b2 = b.reshape(NUM_EXPERTS, 1)
    grid = (TOKENS // BLOCK_T,)
    out, idx = pl.pallas_call(
        _router_kernel,
        grid=grid,
        in_specs=[
            pl.BlockSpec((BLOCK_T, EMBED), lambda i: (i, 0)),
            pl.BlockSpec((NUM_EXPERTS, EMBED), lambda i: (0, 0)),
            pl.BlockSpec((NUM_EXPERTS, 1), lambda i: (0, 0)),
        ],
        out_specs=[
            pl.BlockSpec((BLOCK_T, NUM_EXPERTS), lambda i: (i, 0)),
            pl.BlockSpec((BLOCK_T, ACTIVE_EXPERTS), lambda i: (i, 0)),
        ],
        out_shape=[
            jax.ShapeDtypeStruct((TOKENS, NUM_EXPERTS), jnp.float32),
            jax.ShapeDtypeStruct((TOKENS, ACTIVE_EXPERTS), jnp.int32),
        ],
    )(inputs, W, b2)
    return (out, idx)
